# baseline (speedup 1.0000x reference)
import jax
import jax.numpy as jnp
from jax import lax
from jax.experimental import pallas as pl
from jax.experimental.pallas import tpu as pltpu
from jax.experimental.pallas import tpu_sc as plsc

N = 10000
E = 320000
D = 128
NP = 10240
NC = 2
NS = 16
B = 128
NH = 2
H = 40
NCH = NH * H
BD = 128
NCHD = 40
EP = NC * NS * NCH * B
RPS = NP // NS
DCOL = 16
BLK = 1024

_mesh = plsc.VectorSubcoreMesh(
    core_axis_name="c", subcore_axis_name="s", num_cores=NC, num_subcores=NS)


def _deg_body(dst_hbm, dcnt_out, dst_v, deg_t):
    c = 1 - lax.axis_index("c")
    s = lax.axis_index("s")
    pltpu.sync_copy(dst_hbm.at[c, s], dst_v)

    zero16 = jnp.zeros((16,), jnp.float32)

    def _z(i, carry):
        deg_t[pl.ds(i * 16, 16)] = zero16
        return carry

    lax.fori_loop(0, NP // 16, _z, 0)

    one16 = jnp.ones((16,), jnp.float32)

    def _cnt(i, carry):
        for k in range(B // 16):
            idx = dst_v[i, pl.ds(k * 16, 16)]
            plsc.addupdate_scatter(deg_t, [idx], one16)
        return carry

    lax.fori_loop(0, NCH, _cnt, 0)
    pltpu.sync_copy(deg_t, dcnt_out.at[c, s])


_deg = pl.kernel(
    _deg_body,
    out_type=[jax.ShapeDtypeStruct((NC, NS, NP), jnp.float32)],
    mesh=_mesh,
    scratch_types=[
        pltpu.VMEM((NCH, B), jnp.int32),
        pltpu.VMEM((NP,), jnp.float32),
    ],
    compiler_params=pltpu.CompilerParams(needs_layout_passes=False),
    name="deg_count")


def _seg_body(z_hbm, src_hbm, dst_hbm, acc_out, src_v, dst_v, rows_v, acc_s, sem0, sem1):
    if True:
        c = 1 - lax.axis_index("c")
        s = lax.axis_index("s")

        zero16 = jnp.zeros((16,), jnp.float32)

        def _zrow(i, carry):
            for j in range(D // 16):
                rows_v[0, i, pl.ds(j * 16, 16)] = zero16
            return carry

        lax.fori_loop(0, B, _zrow, 0)
        for k in range(RPS // B):
            pltpu.async_copy(rows_v.at[0], acc_s.at[pl.ds(s * RPS + k * B, B)], sem0)
        for k in range(RPS // B):
            pltpu.make_async_copy(
                rows_v.at[0], acc_s.at[pl.ds(s * RPS + k * B, B)], sem0).wait()

        plsc.subcore_barrier()

        for half in range(NH):
            pltpu.sync_copy(src_hbm.at[c, s, half], src_v)
            pltpu.sync_copy(dst_hbm.at[c, s, half], dst_v)
            pltpu.async_copy(z_hbm.at[src_v.at[0]], rows_v.at[0], sem0)

            def _pair(t, carry):
                j0 = 2 * t
                pltpu.async_copy(z_hbm.at[src_v.at[j0 + 1]], rows_v.at[1], sem1)
                pltpu.make_async_copy(z_hbm.at[src_v.at[j0]], rows_v.at[0], sem0).wait()
                pltpu.sync_copy(rows_v.at[0], acc_s.at[dst_v.at[j0]], add=True)
                pltpu.async_copy(
                    z_hbm.at[src_v.at[jnp.minimum(j0 + 2, H - 1)]], rows_v.at[0], sem0)
                pltpu.make_async_copy(z_hbm.at[src_v.at[j0 + 1]], rows_v.at[1], sem1).wait()
                pltpu.sync_copy(rows_v.at[1], acc_s.at[dst_v.at[j0 + 1]], add=True)
                return carry

            lax.fori_loop(0, H // 2, _pair, 0)
            pltpu.make_async_copy(z_hbm.at[src_v.at[0]], rows_v.at[0], sem0).wait()

        plsc.subcore_barrier()
        for k in range(RPS // B):
            pltpu.sync_copy(acc_s.at[pl.ds(s * RPS + k * B, B)], rows_v.at[k % 2])
            pltpu.sync_copy(rows_v.at[k % 2], acc_out.at[c, pl.ds(s * RPS + k * B, B)])

_SEG_SCRATCH = [
    pltpu.VMEM((H, B), jnp.int32),
    pltpu.VMEM((H, B), jnp.int32),
    pltpu.VMEM((2, B, D), jnp.float32),
    pltpu.VMEM_SHARED((NP, D), jnp.float32),
    pltpu.SemaphoreType.DMA,
    pltpu.SemaphoreType.DMA,
]

_seg = pl.kernel(
    _seg_body,
    out_type=[jax.ShapeDtypeStruct((NC, NP, D), jnp.float32)],
    mesh=_mesh,
    scratch_types=_SEG_SCRATCH,
    name="seg_sum")


def _pre_body(x_ref, wl_ref, wr_ref, bl_ref, z_ref, r_ref):
    xb = x_ref[...]
    z_ref[...] = jnp.dot(xb, wl_ref[...], preferred_element_type=jnp.float32)
    r_ref[...] = jnp.dot(xb, wr_ref[...], preferred_element_type=jnp.float32) + bl_ref[...]


def _mid_body(acc_ref, cnt_ref, r1_ref, wl_ref, wr_ref, bl_ref, z_ref, r_ref):
    agg = acc_ref[0] + acc_ref[1]
    deg = jnp.sum(cnt_ref[...], axis=0)[:, None]
    h1 = jnp.maximum(agg / jnp.maximum(deg, 1.0) + r1_ref[...], 0.0)
    z_ref[...] = jnp.dot(h1, wl_ref[...], preferred_element_type=jnp.float32)
    r_ref[...] = jnp.dot(h1, wr_ref[...], preferred_element_type=jnp.float32) + bl_ref[...]


def _fin_body(acc_ref, cnt_ref, r2_ref, o_ref):
    agg = acc_ref[0] + acc_ref[1]
    deg = jnp.sum(cnt_ref[...], axis=0)[:, None]
    o_ref[...] = jax.nn.sigmoid(agg / jnp.maximum(deg, 1.0) + r2_ref[...])


_row_spec = pl.BlockSpec((BLK, D), lambda i: (i, 0))
_w_spec = pl.BlockSpec((D, D), lambda i: (0, 0))
_b_spec = pl.BlockSpec((1, D), lambda i: (0, 0))
_acc_spec = pl.BlockSpec((NC, BLK, D), lambda i: (0, i, 0))
_cnt_spec = pl.BlockSpec((NC * NS, BLK), lambda i: (0, i))
_rows_out = jax.ShapeDtypeStruct((NP, D), jnp.float32)

_pre = pl.pallas_call(
    _pre_body, grid=(NP // BLK,),
    in_specs=[_row_spec, _w_spec, _w_spec, _b_spec],
    out_specs=[_row_spec, _row_spec],
    out_shape=[_rows_out, _rows_out],
)

_mid = pl.pallas_call(
    _mid_body, grid=(NP // BLK,),
    in_specs=[_acc_spec, _cnt_spec, _row_spec, _w_spec, _w_spec, _b_spec],
    out_specs=[_row_spec, _row_spec],
    out_shape=[_rows_out, _rows_out],
)

_fin = pl.pallas_call(
    _fin_body, grid=(NP // BLK,),
    in_specs=[_acc_spec, _cnt_spec, _row_spec],
    out_specs=_row_spec,
    out_shape=_rows_out,
)


def kernel(x, edge_index, edge_attr, Wl1, bl1, Wr1, Wl2, bl2, Wr2):
    del edge_attr
    xp = jnp.zeros((NP, D), jnp.float32).at[:N].set(x)
    pad = jnp.full((EP - E,), NP - 1, jnp.int32)
    srcp = jnp.concatenate([edge_index[0], pad]).reshape(NC, NS, NH, H, B)
    dstp = jnp.concatenate([edge_index[1], pad]).reshape(NC, NS, NH, H, B)
    dstp_d = dstp.reshape(NC, NS, NCH, B)

    (cnt,) = _deg(dstp_d)
    cnt = cnt.reshape(NC * NS, NP)
    z1, r1 = _pre(xp, Wl1.T, Wr1.T, bl1.reshape(1, D))
    (acc1,) = _seg(z1, srcp, dstp)
    z2, r2 = _mid(acc1, cnt, r1, Wl2.T, Wr2.T, bl2.reshape(1, D))
    (acc2,) = _seg(z2, srcp, dstp)
    out = _fin(acc2, cnt, r2)
    return out[:N]


# speedup vs baseline: 3.0309x; 3.0309x over previous
import jax
import jax.numpy as jnp
from jax import lax
from jax.experimental import pallas as pl
from jax.experimental.pallas import tpu as pltpu
from jax.experimental.pallas import tpu_sc as plsc

N = 10000
E = 320000
D = 128
NP = 10240
NC = 2
NS = 16
B = 128
NH = 2
H = 40
NCH = NH * H
BD = 128
NCHD = 40
EP = NC * NS * NCH * B
RPS = NP // NS
DCOL = 16
BLK = 1024

_mesh = plsc.VectorSubcoreMesh(
    core_axis_name="c", subcore_axis_name="s", num_cores=NC, num_subcores=NS)


def _deg_body(dst_hbm, dcnt_out, dst_v, deg_t):
    c = lax.axis_index("c")
    s = lax.axis_index("s")
    pltpu.sync_copy(dst_hbm.at[c, s], dst_v)

    zero16 = jnp.zeros((16,), jnp.float32)

    def _z(i, carry):
        deg_t[pl.ds(i * 16, 16)] = zero16
        return carry

    lax.fori_loop(0, NP // 16, _z, 0)

    one16 = jnp.ones((16,), jnp.float32)

    def _cnt(i, carry):
        for k in range(B // 16):
            idx = dst_v[i, pl.ds(k * 16, 16)]
            plsc.addupdate_scatter(deg_t, [idx], one16)
        return carry

    lax.fori_loop(0, NCH, _cnt, 0)
    pltpu.sync_copy(deg_t, dcnt_out.at[c, s])


_deg = pl.kernel(
    _deg_body,
    out_type=[jax.ShapeDtypeStruct((NC, NS, NP), jnp.float32)],
    mesh=_mesh,
    scratch_types=[
        pltpu.VMEM((NCH, B), jnp.int32),
        pltpu.VMEM((NP,), jnp.float32),
    ],
    compiler_params=pltpu.CompilerParams(needs_layout_passes=False),
    name="deg_count")


def _seg_body(z_hbm, src_hbm, dst_hbm, acc_out, src_v, dst_v, rows_v, acc_s, sem0, sem1):
    if True:
        c = lax.axis_index("c")
        s = lax.axis_index("s")

        zero16 = jnp.zeros((16,), jnp.float32)

        def _zrow(i, carry):
            for j in range(D // 16):
                rows_v[0, i, pl.ds(j * 16, 16)] = zero16
            return carry

        lax.fori_loop(0, B, _zrow, 0)
        for k in range(RPS // B):
            pltpu.async_copy(rows_v.at[0], acc_s.at[pl.ds(s * RPS + k * B, B)], sem0)
        for k in range(RPS // B):
            pltpu.make_async_copy(
                rows_v.at[0], acc_s.at[pl.ds(s * RPS + k * B, B)], sem0).wait()

        plsc.subcore_barrier()

        for half in range(NH):
            pltpu.sync_copy(src_hbm.at[c, s, half], src_v)
            pltpu.sync_copy(dst_hbm.at[c, s, half], dst_v)
            pltpu.async_copy(z_hbm.at[src_v.at[0]], rows_v.at[0], sem0)

            def _pair(t, carry):
                j0 = 2 * t
                pltpu.async_copy(z_hbm.at[src_v.at[j0 + 1]], rows_v.at[1], sem1)
                pltpu.make_async_copy(z_hbm.at[src_v.at[j0]], rows_v.at[0], sem0).wait()
                pltpu.sync_copy(rows_v.at[0], acc_s.at[dst_v.at[j0]], add=True)
                pltpu.async_copy(
                    z_hbm.at[src_v.at[jnp.minimum(j0 + 2, H - 1)]], rows_v.at[0], sem0)
                pltpu.make_async_copy(z_hbm.at[src_v.at[j0 + 1]], rows_v.at[1], sem1).wait()
                pltpu.sync_copy(rows_v.at[1], acc_s.at[dst_v.at[j0 + 1]], add=True)
                return carry

            lax.fori_loop(0, H // 2, _pair, 0)
            pltpu.make_async_copy(z_hbm.at[src_v.at[0]], rows_v.at[0], sem0).wait()

        plsc.subcore_barrier()
        for k in range(RPS // B):
            pltpu.sync_copy(acc_s.at[pl.ds(s * RPS + k * B, B)], rows_v.at[k % 2])
            pltpu.sync_copy(rows_v.at[k % 2], acc_out.at[c, pl.ds(s * RPS + k * B, B)])

_SEG_SCRATCH = [
    pltpu.VMEM((H, B), jnp.int32),
    pltpu.VMEM((H, B), jnp.int32),
    pltpu.VMEM((2, B, D), jnp.float32),
    pltpu.VMEM_SHARED((NP, D), jnp.float32),
    pltpu.SemaphoreType.DMA,
    pltpu.SemaphoreType.DMA,
]

_seg = pl.kernel(
    _seg_body,
    out_type=[jax.ShapeDtypeStruct((NC, NP, D), jnp.float32)],
    mesh=_mesh,
    scratch_types=_SEG_SCRATCH,
    name="seg_sum")


def _pre_body(x_ref, wl_ref, wr_ref, bl_ref, z_ref, r_ref):
    xb = x_ref[...]
    z_ref[...] = jnp.dot(xb, wl_ref[...], preferred_element_type=jnp.float32)
    r_ref[...] = jnp.dot(xb, wr_ref[...], preferred_element_type=jnp.float32) + bl_ref[...]


def _mid_body(acc_ref, cnt_ref, r1_ref, wl_ref, wr_ref, bl_ref, z_ref, r_ref):
    agg = acc_ref[0] + acc_ref[1]
    deg = jnp.sum(cnt_ref[...], axis=0)[:, None]
    h1 = jnp.maximum(agg / jnp.maximum(deg, 1.0) + r1_ref[...], 0.0)
    z_ref[...] = jnp.dot(h1, wl_ref[...], preferred_element_type=jnp.float32)
    r_ref[...] = jnp.dot(h1, wr_ref[...], preferred_element_type=jnp.float32) + bl_ref[...]


def _fin_body(acc_ref, cnt_ref, r2_ref, o_ref):
    agg = acc_ref[0] + acc_ref[1]
    deg = jnp.sum(cnt_ref[...], axis=0)[:, None]
    o_ref[...] = jax.nn.sigmoid(agg / jnp.maximum(deg, 1.0) + r2_ref[...])


_row_spec = pl.BlockSpec((BLK, D), lambda i: (i, 0))
_w_spec = pl.BlockSpec((D, D), lambda i: (0, 0))
_b_spec = pl.BlockSpec((1, D), lambda i: (0, 0))
_acc_spec = pl.BlockSpec((NC, BLK, D), lambda i: (0, i, 0))
_cnt_spec = pl.BlockSpec((NC * NS, BLK), lambda i: (0, i))
_rows_out = jax.ShapeDtypeStruct((NP, D), jnp.float32)

_pre = pl.pallas_call(
    _pre_body, grid=(NP // BLK,),
    in_specs=[_row_spec, _w_spec, _w_spec, _b_spec],
    out_specs=[_row_spec, _row_spec],
    out_shape=[_rows_out, _rows_out],
)

_mid = pl.pallas_call(
    _mid_body, grid=(NP // BLK,),
    in_specs=[_acc_spec, _cnt_spec, _row_spec, _w_spec, _w_spec, _b_spec],
    out_specs=[_row_spec, _row_spec],
    out_shape=[_rows_out, _rows_out],
)

_fin = pl.pallas_call(
    _fin_body, grid=(NP // BLK,),
    in_specs=[_acc_spec, _cnt_spec, _row_spec],
    out_specs=_row_spec,
    out_shape=_rows_out,
)


def kernel(x, edge_index, edge_attr, Wl1, bl1, Wr1, Wl2, bl2, Wr2):
    del edge_attr
    xp = jnp.zeros((NP, D), jnp.float32).at[:N].set(x)
    pad = N + jnp.arange(EP - E, dtype=jnp.int32) % (NP - N)
    srcp = jnp.concatenate([edge_index[0], pad]).reshape(NC, NS, NH, H, B)
    dstp = jnp.concatenate([edge_index[1], pad]).reshape(NC, NS, NH, H, B)
    dstp_d = dstp.reshape(NC, NS, NCH, B)

    (cnt,) = _deg(dstp_d)
    cnt = cnt.reshape(NC * NS, NP)
    z1, r1 = _pre(xp, Wl1.T, Wr1.T, bl1.reshape(1, D))
    (acc1,) = _seg(z1, srcp, dstp)
    z2, r2 = _mid(acc1, cnt, r1, Wl2.T, Wr2.T, bl2.reshape(1, D))
    (acc2,) = _seg(z2, srcp, dstp)
    out = _fin(acc2, cnt, r2)
    return out[:N]


# speedup vs baseline: 3.0782x; 1.0156x over previous
import jax
import jax.numpy as jnp
from jax import lax
from jax.experimental import pallas as pl
from jax.experimental.pallas import tpu as pltpu
from jax.experimental.pallas import tpu_sc as plsc

N = 10000
E = 320000
D = 128
NP = 10240
NC = 2
NS = 16
B = 128
NH = 2
H = 40
NCH = NH * H
BD = 128
NCHD = 40
EP = NC * NS * NCH * B
RPS = NP // NS
DCOL = 16
BLK = 1024

_mesh = plsc.VectorSubcoreMesh(
    core_axis_name="c", subcore_axis_name="s", num_cores=NC, num_subcores=NS)


def _deg_body(dst_hbm, dcnt_out, dst_v, deg_t):
    c = lax.axis_index("c")
    s = lax.axis_index("s")
    pltpu.sync_copy(dst_hbm.at[c, s], dst_v)

    zero16 = jnp.zeros((16,), jnp.float32)

    def _z(i, carry):
        deg_t[pl.ds(i * 16, 16)] = zero16
        return carry

    lax.fori_loop(0, NP // 16, _z, 0)

    one16 = jnp.ones((16,), jnp.float32)

    def _cnt(i, carry):
        for k in range(B // 16):
            idx = dst_v[i, pl.ds(k * 16, 16)]
            plsc.addupdate_scatter(deg_t, [idx], one16)
        return carry

    lax.fori_loop(0, NCH, _cnt, 0)
    pltpu.sync_copy(deg_t, dcnt_out.at[c, s])


_deg = pl.kernel(
    _deg_body,
    out_type=[jax.ShapeDtypeStruct((NC, NS, NP), jnp.float32)],
    mesh=_mesh,
    scratch_types=[
        pltpu.VMEM((NCH, B), jnp.int32),
        pltpu.VMEM((NP,), jnp.float32),
    ],
    compiler_params=pltpu.CompilerParams(needs_layout_passes=False),
    name="deg_count")


def _seg_body(z_hbm, src_hbm, dst_hbm, acc_out, src_v, dst_v, rows_v, acc_s, sem0, sem1):
    if True:
        c = lax.axis_index("c")
        s = lax.axis_index("s")

        zero16 = jnp.zeros((16,), jnp.float32)

        def _zrow(i, carry):
            for j in range(D // 16):
                rows_v[0, i, pl.ds(j * 16, 16)] = zero16
            return carry

        lax.fori_loop(0, B, _zrow, 0)
        for k in range(RPS // B):
            pltpu.async_copy(rows_v.at[0], acc_s.at[pl.ds(s * RPS + k * B, B)], sem0)
        for k in range(RPS // B):
            pltpu.make_async_copy(
                rows_v.at[0], acc_s.at[pl.ds(s * RPS + k * B, B)], sem0).wait()

        plsc.subcore_barrier()

        for half in range(NH):
            pltpu.sync_copy(src_hbm.at[c, s, half], src_v)
            pltpu.sync_copy(dst_hbm.at[c, s, half], dst_v)
            pltpu.async_copy(z_hbm.at[src_v.at[0]], rows_v.at[0], sem0)

            def _pair(t, carry):
                j0 = 2 * t
                pltpu.async_copy(z_hbm.at[src_v.at[j0 + 1]], rows_v.at[1], sem1)
                pltpu.make_async_copy(z_hbm.at[src_v.at[j0]], rows_v.at[0], sem0).wait()
                pltpu.sync_copy(rows_v.at[0], acc_s.at[dst_v.at[j0]], add=True)
                pltpu.async_copy(
                    z_hbm.at[src_v.at[jnp.minimum(j0 + 2, H - 1)]], rows_v.at[0], sem0)
                pltpu.make_async_copy(z_hbm.at[src_v.at[j0 + 1]], rows_v.at[1], sem1).wait()
                pltpu.sync_copy(rows_v.at[1], acc_s.at[dst_v.at[j0 + 1]], add=True)
                return carry

            lax.fori_loop(0, H // 2, _pair, 0)
            pltpu.make_async_copy(z_hbm.at[src_v.at[0]], rows_v.at[0], sem0).wait()

        plsc.subcore_barrier()
        for k in range(RPS // B):
            pltpu.sync_copy(acc_s.at[pl.ds(s * RPS + k * B, B)], rows_v.at[k % 2])
            pltpu.sync_copy(rows_v.at[k % 2], acc_out.at[c, pl.ds(s * RPS + k * B, B)])

_SEG_SCRATCH = [
    pltpu.VMEM((H, B), jnp.int32),
    pltpu.VMEM((H, B), jnp.int32),
    pltpu.VMEM((2, B, D), jnp.float32),
    pltpu.VMEM_SHARED((NP, D), jnp.float32),
    pltpu.SemaphoreType.DMA,
    pltpu.SemaphoreType.DMA,
]

_seg = pl.kernel(
    _seg_body,
    out_type=[jax.ShapeDtypeStruct((NC, NP, D), jnp.float32)],
    mesh=_mesh,
    scratch_types=_SEG_SCRATCH,
    name="seg_sum")


def _pre_body(x_ref, wl_ref, wr_ref, bl_ref, z_ref, r_ref):
    xb = x_ref[...]
    z_ref[...] = jnp.dot(xb, wl_ref[...], preferred_element_type=jnp.float32)
    r_ref[...] = jnp.dot(xb, wr_ref[...], preferred_element_type=jnp.float32) + bl_ref[...]


def _mid_body(acc_ref, cnt_ref, r1_ref, wl_ref, wr_ref, bl_ref, z_ref, r_ref):
    agg = acc_ref[0] + acc_ref[1]
    deg = jnp.sum(cnt_ref[...], axis=0)[:, None]
    h1 = jnp.maximum(agg / jnp.maximum(deg, 1.0) + r1_ref[...], 0.0)
    z_ref[...] = jnp.dot(h1, wl_ref[...], preferred_element_type=jnp.float32)
    r_ref[...] = jnp.dot(h1, wr_ref[...], preferred_element_type=jnp.float32) + bl_ref[...]


def _fin_body(acc_ref, cnt_ref, r2_ref, o_ref):
    agg = acc_ref[0] + acc_ref[1]
    deg = jnp.sum(cnt_ref[...], axis=0)[:, None]
    o_ref[...] = jax.nn.sigmoid(agg / jnp.maximum(deg, 1.0) + r2_ref[...])


_row_spec = pl.BlockSpec((BLK, D), lambda i: (i, 0))
_w_spec = pl.BlockSpec((D, D), lambda i: (0, 0))
_b_spec = pl.BlockSpec((1, D), lambda i: (0, 0))
_acc_spec = pl.BlockSpec((NC, BLK, D), lambda i: (0, i, 0))
_cnt_spec = pl.BlockSpec((NC * NS, BLK), lambda i: (0, i))
_rows_out = jax.ShapeDtypeStruct((NP, D), jnp.float32)

_pre = pl.pallas_call(
    _pre_body, grid=(NP // BLK,),
    in_specs=[_row_spec, _w_spec, _w_spec, _b_spec],
    out_specs=[_row_spec, _row_spec],
    out_shape=[_rows_out, _rows_out],
)

_mid = pl.pallas_call(
    _mid_body, grid=(NP // BLK,),
    in_specs=[_acc_spec, _cnt_spec, _row_spec, _w_spec, _w_spec, _b_spec],
    out_specs=[_row_spec, _row_spec],
    out_shape=[_rows_out, _rows_out],
)

_fin = pl.pallas_call(
    _fin_body, grid=(NP // BLK,),
    in_specs=[_acc_spec, _cnt_spec, _row_spec],
    out_specs=_row_spec,
    out_shape=_rows_out,
)


def kernel(x, edge_index, edge_attr, Wl1, bl1, Wr1, Wl2, bl2, Wr2):
    del edge_attr
    xp = jnp.zeros((NP, D), jnp.float32).at[:N].set(x)
    pad = N + jnp.arange(EP - E, dtype=jnp.int32) % (NP - N)
    eip = jnp.concatenate([edge_index, jnp.tile(pad, (2, 1))], axis=1)
    srcp = eip[0].reshape(NC, NS, NH, H, B)
    dstp = eip[1].reshape(NC, NS, NH, H, B)
    dstp_d = dstp.reshape(NC, NS, NCH, B)

    (cnt,) = _deg(dstp_d)
    cnt = cnt.reshape(NC * NS, NP)
    z1, r1 = _pre(xp, Wl1.T, Wr1.T, bl1.reshape(1, D))
    (acc1,) = _seg(z1, srcp, dstp)
    z2, r2 = _mid(acc1, cnt, r1, Wl2.T, Wr2.T, bl2.reshape(1, D))
    (acc2,) = _seg(z2, srcp, dstp)
    return _fin(acc2, cnt, r2)[:N]
